# baseline (speedup 1.0000x reference)
import jax
import jax.numpy as jnp
from jax.experimental import pallas as pl
from jax.experimental.pallas import tpu as pltpu

_BC = 16


def _pool_body(x_ref, o_ref):
    x = x_ref[...]
    rm = jnp.maximum(x[:, :, :-1, :], x[:, :, 1:, :])
    o_ref[...] = jnp.maximum(rm[:, :, :, :-1], rm[:, :, :, 1:])


def kernel(x):
    N, C, H, W = x.shape
    return pl.pallas_call(
        _pool_body,
        grid=(N, C // _BC),
        in_specs=[pl.BlockSpec((1, _BC, H, W), lambda n, c: (n, c, 0, 0))],
        out_specs=pl.BlockSpec((1, _BC, H - 1, W - 1),
                               lambda n, c: (n, c, 0, 0)),
        out_shape=jax.ShapeDtypeStruct((N, C, H - 1, W - 1), x.dtype),
        compiler_params=pltpu.CompilerParams(
            dimension_semantics=("parallel", "parallel"),
        ),
    )(x)


# speedup vs baseline: 1.0127x; 1.0127x over previous
import jax
import jax.numpy as jnp
from jax.experimental import pallas as pl
from jax.experimental.pallas import tpu as pltpu

_BC = 32


def _pool_body(x_ref, o_ref):
    x = x_ref[...]
    rm = jnp.maximum(x[:, :, :-1, :], x[:, :, 1:, :])
    o_ref[...] = jnp.maximum(rm[:, :, :, :-1], rm[:, :, :, 1:])


def kernel(x):
    N, C, H, W = x.shape
    return pl.pallas_call(
        _pool_body,
        grid=(N, C // _BC),
        in_specs=[pl.BlockSpec((1, _BC, H, W), lambda n, c: (n, c, 0, 0))],
        out_specs=pl.BlockSpec((1, _BC, H - 1, W - 1),
                               lambda n, c: (n, c, 0, 0)),
        out_shape=jax.ShapeDtypeStruct((N, C, H - 1, W - 1), x.dtype),
        compiler_params=pltpu.CompilerParams(
            dimension_semantics=("parallel", "parallel"),
            vmem_limit_bytes=100 * 1024 * 1024,
        ),
    )(x)


# speedup vs baseline: 1.1989x; 1.1839x over previous
import jax
import jax.numpy as jnp
from jax.experimental import pallas as pl
from jax.experimental.pallas import tpu as pltpu

_BC = 32


def _pool_body(x_ref, o_ref):
    x = x_ref[...]
    rm = jnp.maximum(x[:, :-1, :], x[:, 1:, :])
    o_ref[...] = jnp.maximum(rm[:, :, :-1], rm[:, :, 1:])


def kernel(x):
    N, C, H, W = x.shape
    nc = N * C
    xf = x.reshape(nc, H, W)
    out = pl.pallas_call(
        _pool_body,
        grid=(nc // _BC,),
        in_specs=[pl.BlockSpec((_BC, H, W), lambda i: (i, 0, 0))],
        out_specs=pl.BlockSpec((_BC, H - 1, W - 1), lambda i: (i, 0, 0)),
        out_shape=jax.ShapeDtypeStruct((nc, H - 1, W - 1), x.dtype),
        compiler_params=pltpu.CompilerParams(
            dimension_semantics=("parallel",),
            vmem_limit_bytes=100 * 1024 * 1024,
        ),
    )(xf)
    return out.reshape(N, C, H - 1, W - 1)
